# baseline (speedup 1.0000x reference)
import functools

import jax
import jax.numpy as jnp
from jax import lax
from jax.experimental import pallas as pl
from jax.experimental.pallas import tpu as pltpu
from jax.experimental.pallas import tpu_sc as plsc

_V = 8192
_D = 8192
_B = 4096
_NW = 32
_BPW = _B // _NW
_G = _BPW // 16

_mesh = plsc.VectorSubcoreMesh(core_axis_name="c", subcore_axis_name="s")


@functools.partial(
    pl.kernel,
    mesh=_mesh,
    out_type=jax.ShapeDtypeStruct((_B, _D), jnp.float32),
    scratch_types=[
        pltpu.VMEM((_BPW,), jnp.int32),
        pltpu.SemaphoreType.DMA,
    ],
)
def _sc_gather(table_hbm, idx_hbm, out_hbm, idx_v, sem):
    wid = lax.axis_index("s") * 2 + lax.axis_index("c")
    base = wid * _BPW
    pltpu.sync_copy(idx_hbm.at[pl.ds(base, _BPW)], idx_v)

    def fire(g, _):
        vec = idx_v[pl.ds(g * 16, 16)]
        for l in range(16):
            row = vec[l]
            pltpu.async_copy(table_hbm.at[pl.ds(row, 1)],
                             out_hbm.at[pl.ds(base + g * 16 + l, 1)], sem)
        return 0

    lax.fori_loop(0, _G, fire, 0)

    def drain(c, _):
        pltpu.make_async_copy(table_hbm.at[pl.ds(0, 1)],
                              out_hbm.at[pl.ds(base, 1)], sem).wait()
        return 0

    lax.fori_loop(0, _BPW, drain, 0)


def kernel(idx, table):
    idx1 = idx.reshape(_B).astype(jnp.int32)
    out = _sc_gather(table, idx1)
    return out.reshape(idx.shape[0], idx.shape[1], _D)


# speedup vs baseline: 15.5548x; 15.5548x over previous
import functools

import jax
import jax.numpy as jnp
from jax import lax
from jax.experimental import pallas as pl
from jax.experimental.pallas import tpu as pltpu
from jax.experimental.pallas import tpu_sc as plsc

_V = 8192
_D = 8192
_B = 4096
_NW = 32
_R = 4
_CPW = (_B // _NW) // _R

_mesh = plsc.VectorSubcoreMesh(core_axis_name="c", subcore_axis_name="s")


@functools.partial(
    pl.kernel,
    mesh=_mesh,
    out_type=jax.ShapeDtypeStruct((_B // _R, _R, _D), jnp.float32),
    scratch_types=[
        pltpu.VMEM((_CPW, _R), jnp.int32),
        pltpu.VMEM((_R, _D), jnp.float32),
        pltpu.VMEM((_R, _D), jnp.float32),
        pltpu.SemaphoreType.DMA,
        pltpu.SemaphoreType.DMA,
        pltpu.SemaphoreType.DMA,
        pltpu.SemaphoreType.DMA,
    ],
)
def _sc_gather(table_hbm, idx_hbm, out_hbm, idx_v, buf0, buf1,
               gsem0, gsem1, ssem0, ssem1):
    wid = lax.axis_index("s") * 2 + lax.axis_index("c")
    pltpu.sync_copy(idx_hbm.at[wid], idx_v)
    cbase = wid * _CPW

    pltpu.async_copy(table_hbm.at[idx_v.at[0]], buf0, gsem0)
    pltpu.async_copy(table_hbm.at[idx_v.at[1]], buf1, gsem1)

    def body(i, _):
        c = i * 2
        pltpu.make_async_copy(table_hbm.at[idx_v.at[c]], buf0, gsem0).wait()
        pltpu.async_copy(buf0, out_hbm.at[cbase + c], ssem0)

        pltpu.make_async_copy(table_hbm.at[idx_v.at[c + 1]], buf1, gsem1).wait()
        pltpu.async_copy(buf1, out_hbm.at[cbase + c + 1], ssem1)

        @pl.when(i < _CPW // 2 - 1)
        def _():
            pltpu.make_async_copy(buf0, out_hbm.at[cbase + c], ssem0).wait()
            pltpu.async_copy(table_hbm.at[idx_v.at[c + 2]], buf0, gsem0)
            pltpu.make_async_copy(buf1, out_hbm.at[cbase + c + 1], ssem1).wait()
            pltpu.async_copy(table_hbm.at[idx_v.at[c + 3]], buf1, gsem1)

        return 0

    lax.fori_loop(0, _CPW // 2, body, 0)

    last = cbase + _CPW - 2
    pltpu.make_async_copy(buf0, out_hbm.at[last], ssem0).wait()
    pltpu.make_async_copy(buf1, out_hbm.at[last + 1], ssem1).wait()


def kernel(idx, table):
    idx3 = idx.reshape(_NW, _CPW, _R).astype(jnp.int32)
    out = _sc_gather(table, idx3)
    return out.reshape(idx.shape[0], idx.shape[1], _D)


# speedup vs baseline: 34.4006x; 2.2116x over previous
import functools

import jax
import jax.numpy as jnp
from jax import lax
from jax.experimental import pallas as pl
from jax.experimental.pallas import tpu as pltpu
from jax.experimental.pallas import tpu_sc as plsc

_V = 8192
_D = 8192
_B = 4096
_NW = 32
_R = 4
_CPW = (_B // _NW) // _R

_mesh = plsc.VectorSubcoreMesh(core_axis_name="c", subcore_axis_name="s")


@functools.partial(
    pl.kernel,
    mesh=_mesh,
    out_type=jax.ShapeDtypeStruct((_B, _D), jnp.float32),
    scratch_types=[
        pltpu.VMEM((_CPW, _R), jnp.int32),
        pltpu.VMEM((_R, _D), jnp.float32),
        pltpu.VMEM((_R, _D), jnp.float32),
        pltpu.SemaphoreType.DMA,
        pltpu.SemaphoreType.DMA,
        pltpu.SemaphoreType.DMA,
        pltpu.SemaphoreType.DMA,
    ],
)
def _sc_gather(table_hbm, idx_hbm, out_hbm, idx_v, buf0, buf1,
               gsem0, gsem1, ssem0, ssem1):
    wid = lax.axis_index("s") * 2 + lax.axis_index("c")
    pltpu.sync_copy(idx_hbm.at[wid], idx_v)
    rbase = wid * _CPW * _R

    pltpu.async_copy(table_hbm.at[idx_v.at[0]], buf0, gsem0)
    pltpu.async_copy(table_hbm.at[idx_v.at[1]], buf1, gsem1)

    def body(i, _):
        c = i * 2
        pltpu.make_async_copy(table_hbm.at[idx_v.at[c]], buf0, gsem0).wait()
        pltpu.async_copy(buf0, out_hbm.at[pl.ds(rbase + c * _R, _R)], ssem0)

        pltpu.make_async_copy(table_hbm.at[idx_v.at[c + 1]], buf1, gsem1).wait()
        pltpu.async_copy(buf1, out_hbm.at[pl.ds(rbase + (c + 1) * _R, _R)],
                         ssem1)

        @pl.when(i < _CPW // 2 - 1)
        def _():
            pltpu.make_async_copy(
                buf0, out_hbm.at[pl.ds(rbase + c * _R, _R)], ssem0).wait()
            pltpu.async_copy(table_hbm.at[idx_v.at[c + 2]], buf0, gsem0)
            pltpu.make_async_copy(
                buf1, out_hbm.at[pl.ds(rbase + (c + 1) * _R, _R)],
                ssem1).wait()
            pltpu.async_copy(table_hbm.at[idx_v.at[c + 3]], buf1, gsem1)

        return 0

    lax.fori_loop(0, _CPW // 2, body, 0)

    last = rbase + (_CPW - 2) * _R
    pltpu.make_async_copy(buf0, out_hbm.at[pl.ds(last, _R)], ssem0).wait()
    pltpu.make_async_copy(buf1, out_hbm.at[pl.ds(last + _R, _R)], ssem1).wait()


def kernel(idx, table):
    idx3 = idx.reshape(_NW, _CPW, _R).astype(jnp.int32)
    out = _sc_gather(table, idx3)
    return out.reshape(idx.shape[0], idx.shape[1], _D)


# speedup vs baseline: 35.6139x; 1.0353x over previous
import functools

import jax
import jax.numpy as jnp
from jax import lax
from jax.experimental import pallas as pl
from jax.experimental.pallas import tpu as pltpu
from jax.experimental.pallas import tpu_sc as plsc

_V = 8192
_D = 8192
_B = 4096
_NW = 32
_R = 2
_NBUF = 4
_CPW = (_B // _NW) // _R

_mesh = plsc.VectorSubcoreMesh(core_axis_name="c", subcore_axis_name="s")


@functools.partial(
    pl.kernel,
    mesh=_mesh,
    out_type=jax.ShapeDtypeStruct((_B, _D), jnp.float32),
    scratch_types=[
        pltpu.VMEM((_CPW, _R), jnp.int32),
    ] + [pltpu.VMEM((_R, _D), jnp.float32)] * _NBUF
      + [pltpu.SemaphoreType.DMA] * (2 * _NBUF),
)
def _sc_gather(table_hbm, idx_hbm, out_hbm, idx_v, *bufs_and_sems):
    bufs = bufs_and_sems[:_NBUF]
    gsems = bufs_and_sems[_NBUF:2 * _NBUF]
    ssems = bufs_and_sems[2 * _NBUF:]
    wid = lax.axis_index("s") * 2 + lax.axis_index("c")
    pltpu.sync_copy(idx_hbm.at[wid], idx_v)
    rbase = wid * _CPW * _R

    def out_rows(c):
        return out_hbm.at[pl.ds(rbase + c * _R, _R)]

    for j in range(_NBUF):
        pltpu.async_copy(table_hbm.at[idx_v.at[j]], bufs[j], gsems[j])

    def body(i, _):
        c0 = i * _NBUF
        for j in range(_NBUF):
            pltpu.make_async_copy(
                table_hbm.at[idx_v.at[c0 + j]], bufs[j], gsems[j]).wait()
            pltpu.async_copy(bufs[j], out_rows(c0 + j), ssems[j])

        @pl.when(i < _CPW // _NBUF - 1)
        def _():
            for j in range(_NBUF):
                pltpu.make_async_copy(
                    bufs[j], out_rows(c0 + j), ssems[j]).wait()
                pltpu.async_copy(
                    table_hbm.at[idx_v.at[c0 + _NBUF + j]], bufs[j], gsems[j])

        return 0

    lax.fori_loop(0, _CPW // _NBUF, body, 0)

    clast = _CPW - _NBUF
    for j in range(_NBUF):
        pltpu.make_async_copy(bufs[j], out_rows(clast + j), ssems[j]).wait()


def kernel(idx, table):
    idx3 = idx.reshape(_NW, _CPW, _R).astype(jnp.int32)
    out = _sc_gather(table, idx3)
    return out.reshape(idx.shape[0], idx.shape[1], _D)
